# baseline (speedup 1.0000x reference)
import functools

import jax
import jax.numpy as jnp
from jax.experimental import pallas as pl

H = 32
D = 64
CS = 256
TOPK = 4
EPS = 1e-6
NEG = -1e30

BF = jnp.bfloat16
F32 = jnp.float32


def _dot(a, b, dims):
    return jax.lax.dot_general(a, b, dimension_numbers=(dims, ((), ())),
                               preferred_element_type=F32)


def _proj_rope_kern(hs_ref, w_ref, wp_ref, cos_ref, sin_ref, out_ref, *, f32_out):
    x = _dot(hs_ref[...], w_ref[...], ((1,), (0,)))
    xp = _dot(hs_ref[...], wp_ref[...], ((1,), (0,)))
    r = x * cos_ref[...] + xp * sin_ref[...]
    out_ref[...] = r if f32_out else r.astype(BF)


def _proj_rope(hsb, w, wp, cosT, sinT2, f32_out):
    S, HID = hsb.shape
    HD = w.shape[1]
    BR, BN = 512, 1024
    return pl.pallas_call(
        functools.partial(_proj_rope_kern, f32_out=f32_out),
        grid=(S // BR, HD // BN),
        in_specs=[
            pl.BlockSpec((BR, HID), lambda r, n: (r, 0)),
            pl.BlockSpec((HID, BN), lambda r, n: (0, n)),
            pl.BlockSpec((HID, BN), lambda r, n: (0, n)),
            pl.BlockSpec((BR, BN), lambda r, n: (r, n)),
            pl.BlockSpec((BR, BN), lambda r, n: (r, n)),
        ],
        out_specs=pl.BlockSpec((BR, BN), lambda r, n: (r, n)),
        out_shape=jax.ShapeDtypeStruct((S, HD), F32 if f32_out else BF),
    )(hsb, w, wp, cosT, sinT2)


def _proj_kern(hs_ref, w_ref, out_ref):
    out_ref[...] = _dot(hs_ref[...], w_ref[...], ((1,), (0,))).astype(BF)


def _proj(hsb, w):
    S, HID = hsb.shape
    HD = w.shape[1]
    BR, BN = 512, 1024
    return pl.pallas_call(
        _proj_kern,
        grid=(S // BR, HD // BN),
        in_specs=[
            pl.BlockSpec((BR, HID), lambda r, n: (r, 0)),
            pl.BlockSpec((HID, BN), lambda r, n: (0, n)),
        ],
        out_specs=pl.BlockSpec((BR, BN), lambda r, n: (r, n)),
        out_shape=jax.ShapeDtypeStruct((S, HD), BF),
    )(hsb, w)


def _gatepath_kern(hsT_ref, w1_ref, w2_ref, onw_ref, out_ref):
    tT = _dot(w1_ref[...], hsT_ref[...], ((0,), (0,))).astype(BF)
    gT = _dot(w2_ref[...], tT, ((0,), (0,)))
    out_ref[...] = jax.nn.sigmoid(gT) * onw_ref[:, 0:1]


def _gatepath_T(hsTb, w1, w2, onwT):
    HID, S = hsTb.shape
    BN = 512
    return pl.pallas_call(
        _gatepath_kern,
        grid=(S // BN,),
        in_specs=[
            pl.BlockSpec((HID, BN), lambda n: (0, n)),
            pl.BlockSpec((HID, D), lambda n: (0, 0)),
            pl.BlockSpec((D, HID), lambda n: (0, 0)),
            pl.BlockSpec((HID, 128), lambda n: (0, 0)),
        ],
        out_specs=pl.BlockSpec((HID, BN), lambda n: (0, n)),
        out_shape=jax.ShapeDtypeStruct((HID, S), F32),
    )(hsTb, w1, w2, onwT)


def _attn_kern(qb_ref, kf_ref, vb_ref, sg_ref, out_ref, *, S, scale):
    C = S // CS
    qb = qb_ref[:, 0, 0, :]

    kc = jnp.concatenate(
        [jnp.mean(kf_ref[c * CS:(c + 1) * CS, 0, 0, :], axis=0, keepdims=True)
         for c in range(C)], axis=0)
    kcb = kc.astype(BF)
    g = _dot(kcb, qb, ((1,), (1,)))
    c = jax.lax.broadcasted_iota(jnp.int32, (C, S), 0)
    pos = jax.lax.broadcasted_iota(jnp.int32, (C, S), 1)
    cid = pos // CS
    g = jnp.where(pos < (c + 1) * CS, -jnp.inf, g)
    g = jnp.where(cid == c, jnp.inf, g)
    cnt = jnp.zeros((C, S), jnp.int32)
    for cc in range(C):
        cnt = cnt + (g[cc:cc + 1, :] >= g).astype(jnp.int32)
    cand = jnp.where(cnt >= TOPK, g, -jnp.inf)
    thresh = jnp.max(cand, axis=0, keepdims=True)
    bias = jnp.where(g >= thresh, 0.0, NEG)

    ci = jax.lax.broadcasted_iota(jnp.int32, (CS, CS), 0)
    qi = jax.lax.broadcasted_iota(jnp.int32, (CS, CS), 1)
    tri = ci > qi

    for i in range(C):
        q_i = qb[i * CS:(i + 1) * CS, :]
        m = jnp.full((1, CS), NEG, dtype=F32)
        l = jnp.zeros((1, CS), dtype=F32)
        acc = jnp.zeros((D, CS), dtype=F32)
        for j in range(i + 1):
            kj = kf_ref[j * CS:(j + 1) * CS, 0, 0, :].astype(BF)
            vj = vb_ref[j * CS:(j + 1) * CS, 0, 0, :]
            s = _dot(kj, q_i, ((1,), (1,))) * scale
            s = s + bias[j:j + 1, i * CS:(i + 1) * CS]
            if j == i:
                s = jnp.where(tri, NEG, s)
            m_new = jnp.maximum(m, jnp.max(s, axis=0, keepdims=True))
            r = jnp.exp(m - m_new)
            p = jnp.exp(s - m_new)
            l = l * r + jnp.sum(p, axis=0, keepdims=True)
            acc = acc * r + _dot(vj, p.astype(BF), ((0,), (0,)))
            m = m_new
        oT = acc / l
        rms = jax.lax.rsqrt(jnp.mean(oT * oT, axis=0, keepdims=True) + EPS)
        out_ref[:, i * CS:(i + 1) * CS] = \
            (oT * rms * sg_ref[:, i * CS:(i + 1) * CS]).astype(BF)


def _attention(qb4, kf4, vb4, sgT):
    S = qb4.shape[0]
    scale = 1.0 / (D ** 0.5)
    return pl.pallas_call(
        functools.partial(_attn_kern, S=S, scale=scale),
        grid=(H,),
        in_specs=[
            pl.BlockSpec((S, 1, 1, D), lambda h: (0, h, 0, 0)),
            pl.BlockSpec((S, 1, 1, D), lambda h: (0, h, 0, 0)),
            pl.BlockSpec((S, 1, 1, D), lambda h: (0, h, 0, 0)),
            pl.BlockSpec((D, S), lambda h: (h, 0)),
        ],
        out_specs=pl.BlockSpec((D, S), lambda h: (h, 0)),
        out_shape=jax.ShapeDtypeStruct((H * D, S), BF),
    )(qb4, kf4, vb4, sgT)


def _outproj_kern(oT_ref, w_ref, out_ref):
    out_ref[...] = _dot(oT_ref[...], w_ref[...], ((0,), (0,)))


def _outproj(oT, w):
    HD, S = oT.shape
    HID = w.shape[1]
    BN = 512
    return pl.pallas_call(
        _outproj_kern,
        grid=(S // BN, HID // BN),
        in_specs=[
            pl.BlockSpec((HD, BN), lambda r, n: (0, r)),
            pl.BlockSpec((HD, BN), lambda r, n: (0, n)),
        ],
        out_specs=pl.BlockSpec((BN, BN), lambda r, n: (r, n)),
        out_shape=jax.ShapeDtypeStruct((S, HID), F32),
    )(oT, w)


def _swap_halves(w):
    HID = w.shape[0]
    w3 = w.reshape(HID, H, D)
    return jnp.concatenate([w3[:, :, D // 2:], w3[:, :, : D // 2]],
                           axis=-1).reshape(HID, H * D)


def kernel(hidden_states, Wq, Wk, Wv, Wo, Wg1, Wg2, o_norm_w):
    B, S, HID = hidden_states.shape
    hs = hidden_states[0]
    hsb = hs.astype(BF)

    pos = jnp.arange(S)
    inv = 1.0 / (10000.0 ** (jnp.arange(0, D, 2, dtype=F32) / D))
    fr = pos[:, None].astype(F32) * inv[None, :]
    cos = jnp.cos(fr)
    sin = jnp.sin(fr)
    cosT = jnp.tile(jnp.concatenate([cos, cos], axis=1), (1, H))
    sinT2 = jnp.tile(jnp.concatenate([-sin, sin], axis=1), (1, H))

    qb = _proj_rope(hsb, Wq.astype(BF), _swap_halves(Wq).astype(BF),
                    cosT, sinT2, f32_out=False)
    kf = _proj_rope(hsb, Wk.astype(BF), _swap_halves(Wk).astype(BF),
                    cosT, sinT2, f32_out=True)
    vb = _proj(hsb, Wv.astype(BF))
    onwT = jnp.broadcast_to(jnp.tile(o_norm_w, H)[:, None], (H * D, 128))
    sgT = _gatepath_T(hsb.T, Wg1.astype(BF), Wg2.astype(BF), onwT)

    qb4 = qb.reshape(S, H, 1, D)
    kf4 = kf.reshape(S, H, 1, D)
    vb4 = vb.reshape(S, H, 1, D)

    oT = _attention(qb4, kf4, vb4, sgT)
    out = _outproj(oT, Wo.astype(BF))
    return out[None]


# speedup vs baseline: 1.0055x; 1.0055x over previous
import functools

import jax
import jax.numpy as jnp
from jax.experimental import pallas as pl
from jax.experimental.pallas import tpu as pltpu

H = 32
D = 64
CS = 256
TOPK = 4
EPS = 1e-6
NEG = -1e30

BF = jnp.bfloat16
F32 = jnp.float32


def _dot(a, b, dims):
    return jax.lax.dot_general(a, b, dimension_numbers=(dims, ((), ())),
                               preferred_element_type=F32)


def _proj_rope_kern(hs_ref, w_ref, wp_ref, cos_ref, sin_ref, out_ref, *, f32_out):
    x = _dot(hs_ref[...], w_ref[...], ((1,), (0,)))
    xp = _dot(hs_ref[...], wp_ref[...], ((1,), (0,)))
    r = x * cos_ref[...] + xp * sin_ref[...]
    out_ref[...] = r if f32_out else r.astype(BF)


def _proj_rope(hsb, w, wp, cosT, sinT2, f32_out):
    S, HID = hsb.shape
    HD = w.shape[1]
    BR, BN = 512, 1024
    return pl.pallas_call(
        functools.partial(_proj_rope_kern, f32_out=f32_out),
        grid=(S // BR, HD // BN),
        in_specs=[
            pl.BlockSpec((BR, HID), lambda r, n: (r, 0)),
            pl.BlockSpec((HID, BN), lambda r, n: (0, n)),
            pl.BlockSpec((HID, BN), lambda r, n: (0, n)),
            pl.BlockSpec((BR, BN), lambda r, n: (r, n)),
            pl.BlockSpec((BR, BN), lambda r, n: (r, n)),
        ],
        out_specs=pl.BlockSpec((BR, BN), lambda r, n: (r, n)),
        out_shape=jax.ShapeDtypeStruct((S, HD), F32 if f32_out else BF),
        compiler_params=pltpu.CompilerParams(
            dimension_semantics=("parallel", "parallel")),
    )(hsb, w, wp, cosT, sinT2)


def _proj_kern(hs_ref, w_ref, out_ref):
    out_ref[...] = _dot(hs_ref[...], w_ref[...], ((1,), (0,))).astype(BF)


def _proj(hsb, w):
    S, HID = hsb.shape
    HD = w.shape[1]
    BR, BN = 512, 1024
    return pl.pallas_call(
        _proj_kern,
        grid=(S // BR, HD // BN),
        in_specs=[
            pl.BlockSpec((BR, HID), lambda r, n: (r, 0)),
            pl.BlockSpec((HID, BN), lambda r, n: (0, n)),
        ],
        out_specs=pl.BlockSpec((BR, BN), lambda r, n: (r, n)),
        out_shape=jax.ShapeDtypeStruct((S, HD), BF),
        compiler_params=pltpu.CompilerParams(
            dimension_semantics=("parallel", "parallel")),
    )(hsb, w)


def _gatepath_kern(hsT_ref, w1_ref, w2_ref, onw_ref, out_ref):
    tT = _dot(w1_ref[...], hsT_ref[...], ((0,), (0,))).astype(BF)
    gT = _dot(w2_ref[...], tT, ((0,), (0,)))
    out_ref[...] = jax.nn.sigmoid(gT) * onw_ref[:, 0:1]


def _gatepath_T(hsTb, w1, w2, onwT):
    HID, S = hsTb.shape
    BN = 512
    return pl.pallas_call(
        _gatepath_kern,
        grid=(S // BN,),
        in_specs=[
            pl.BlockSpec((HID, BN), lambda n: (0, n)),
            pl.BlockSpec((HID, D), lambda n: (0, 0)),
            pl.BlockSpec((D, HID), lambda n: (0, 0)),
            pl.BlockSpec((HID, 128), lambda n: (0, 0)),
        ],
        out_specs=pl.BlockSpec((HID, BN), lambda n: (0, n)),
        out_shape=jax.ShapeDtypeStruct((HID, S), F32),
        compiler_params=pltpu.CompilerParams(
            dimension_semantics=("parallel",)),
    )(hsTb, w1, w2, onwT)


def _attn_kern(qb_ref, kf_ref, vb_ref, sg_ref, out_ref, *, S, scale):
    C = S // CS
    qb = qb_ref[:, 0, 0, :]

    kc = jnp.concatenate(
        [jnp.mean(kf_ref[c * CS:(c + 1) * CS, 0, 0, :], axis=0, keepdims=True)
         for c in range(C)], axis=0)
    kcb = kc.astype(BF)
    g = _dot(kcb, qb, ((1,), (1,)))
    c = jax.lax.broadcasted_iota(jnp.int32, (C, S), 0)
    pos = jax.lax.broadcasted_iota(jnp.int32, (C, S), 1)
    cid = pos // CS
    g = jnp.where(pos < (c + 1) * CS, -jnp.inf, g)
    g = jnp.where(cid == c, jnp.inf, g)
    cnt = jnp.zeros((C, S), jnp.int32)
    for cc in range(C):
        cnt = cnt + (g[cc:cc + 1, :] >= g).astype(jnp.int32)
    cand = jnp.where(cnt >= TOPK, g, -jnp.inf)
    thresh = jnp.max(cand, axis=0, keepdims=True)
    bias = jnp.where(g >= thresh, 0.0, NEG)

    ci = jax.lax.broadcasted_iota(jnp.int32, (CS, CS), 0)
    qi = jax.lax.broadcasted_iota(jnp.int32, (CS, CS), 1)
    tri = ci > qi

    for i in range(C):
        q_i = qb[i * CS:(i + 1) * CS, :]
        m = jnp.full((1, CS), NEG, dtype=F32)
        l = jnp.zeros((1, CS), dtype=F32)
        acc = jnp.zeros((D, CS), dtype=F32)
        for j in range(i + 1):
            kj = kf_ref[j * CS:(j + 1) * CS, 0, 0, :].astype(BF)
            vj = vb_ref[j * CS:(j + 1) * CS, 0, 0, :]
            s = _dot(kj, q_i, ((1,), (1,))) * scale
            s = s + bias[j:j + 1, i * CS:(i + 1) * CS]
            if j == i:
                s = jnp.where(tri, NEG, s)
            m_new = jnp.maximum(m, jnp.max(s, axis=0, keepdims=True))
            r = jnp.exp(m - m_new)
            p = jnp.exp(s - m_new)
            l = l * r + jnp.sum(p, axis=0, keepdims=True)
            acc = acc * r + _dot(vj, p.astype(BF), ((0,), (0,)))
            m = m_new
        oT = acc / l
        rms = jax.lax.rsqrt(jnp.mean(oT * oT, axis=0, keepdims=True) + EPS)
        out_ref[:, i * CS:(i + 1) * CS] = \
            (oT * rms * sg_ref[:, i * CS:(i + 1) * CS]).astype(BF)


def _attention(qb4, kf4, vb4, sgT):
    S = qb4.shape[0]
    scale = 1.0 / (D ** 0.5)
    return pl.pallas_call(
        functools.partial(_attn_kern, S=S, scale=scale),
        grid=(H,),
        in_specs=[
            pl.BlockSpec((S, 1, 1, D), lambda h: (0, h, 0, 0)),
            pl.BlockSpec((S, 1, 1, D), lambda h: (0, h, 0, 0)),
            pl.BlockSpec((S, 1, 1, D), lambda h: (0, h, 0, 0)),
            pl.BlockSpec((D, S), lambda h: (h, 0)),
        ],
        out_specs=pl.BlockSpec((D, S), lambda h: (h, 0)),
        out_shape=jax.ShapeDtypeStruct((H * D, S), BF),
        compiler_params=pltpu.CompilerParams(
            dimension_semantics=("parallel",)),
    )(qb4, kf4, vb4, sgT)


def _outproj_kern(oT_ref, w_ref, out_ref):
    out_ref[...] = _dot(oT_ref[...], w_ref[...], ((0,), (0,)))


def _outproj(oT, w):
    HD, S = oT.shape
    HID = w.shape[1]
    BN = 512
    return pl.pallas_call(
        _outproj_kern,
        grid=(S // BN, HID // BN),
        in_specs=[
            pl.BlockSpec((HD, BN), lambda r, n: (0, r)),
            pl.BlockSpec((HD, BN), lambda r, n: (0, n)),
        ],
        out_specs=pl.BlockSpec((BN, BN), lambda r, n: (r, n)),
        out_shape=jax.ShapeDtypeStruct((S, HID), F32),
        compiler_params=pltpu.CompilerParams(
            dimension_semantics=("parallel", "parallel")),
    )(oT, w)


def _swap_halves(w):
    HID = w.shape[0]
    w3 = w.reshape(HID, H, D)
    return jnp.concatenate([w3[:, :, D // 2:], w3[:, :, : D // 2]],
                           axis=-1).reshape(HID, H * D)


def kernel(hidden_states, Wq, Wk, Wv, Wo, Wg1, Wg2, o_norm_w):
    B, S, HID = hidden_states.shape
    hs = hidden_states[0]
    hsb = hs.astype(BF)

    pos = jnp.arange(S)
    inv = 1.0 / (10000.0 ** (jnp.arange(0, D, 2, dtype=F32) / D))
    fr = pos[:, None].astype(F32) * inv[None, :]
    cos = jnp.cos(fr)
    sin = jnp.sin(fr)
    cosT = jnp.tile(jnp.concatenate([cos, cos], axis=1), (1, H))
    sinT2 = jnp.tile(jnp.concatenate([-sin, sin], axis=1), (1, H))

    qb = _proj_rope(hsb, Wq.astype(BF), _swap_halves(Wq).astype(BF),
                    cosT, sinT2, f32_out=False)
    kf = _proj_rope(hsb, Wk.astype(BF), _swap_halves(Wk).astype(BF),
                    cosT, sinT2, f32_out=True)
    vb = _proj(hsb, Wv.astype(BF))
    onwT = jnp.broadcast_to(jnp.tile(o_norm_w, H)[:, None], (H * D, 128))
    sgT = _gatepath_T(hsb.T, Wg1.astype(BF), Wg2.astype(BF), onwT)

    qb4 = qb.reshape(S, H, 1, D)
    kf4 = kf.reshape(S, H, 1, D)
    vb4 = vb.reshape(S, H, 1, D)

    oT = _attention(qb4, kf4, vb4, sgT)
    out = _outproj(oT, Wo.astype(BF))
    return out[None]


# speedup vs baseline: 1.2240x; 1.2173x over previous
import functools

import jax
import jax.numpy as jnp
from jax.experimental import pallas as pl
from jax.experimental.pallas import tpu as pltpu

H = 32
D = 64
CS = 256
TOPK = 4
EPS = 1e-6
NEG = -1e30

BF = jnp.bfloat16
F32 = jnp.float32


def _dot(a, b):
    return jax.lax.dot_general(a, b, dimension_numbers=((((a.ndim - 1,), (0,))), ((), ())),
                               preferred_element_type=F32)


def _mm_kern(a_ref, b_ref, out_ref, *, out_bf16):
    r = _dot(a_ref[...], b_ref[...])
    out_ref[...] = r.astype(BF) if out_bf16 else r


def _matmul(a, b, out_bf16, bm=512, bn=512):
    M, K = a.shape
    N = b.shape[1]
    return pl.pallas_call(
        functools.partial(_mm_kern, out_bf16=out_bf16),
        grid=(M // bm, N // bn),
        in_specs=[
            pl.BlockSpec((bm, K), lambda m, n: (m, 0)),
            pl.BlockSpec((K, bn), lambda m, n: (0, n)),
        ],
        out_specs=pl.BlockSpec((bm, bn), lambda m, n: (m, n)),
        out_shape=jax.ShapeDtypeStruct((M, N), BF if out_bf16 else F32),
        compiler_params=pltpu.CompilerParams(
            dimension_semantics=("parallel", "parallel")),
    )(a, b)


def _gatepath_kern(w1T_ref, hsT_ref, w2T_ref, onw_ref, out_ref):
    tT = _dot(w1T_ref[...], hsT_ref[...]).astype(BF)
    gT = _dot(w2T_ref[...], tT)
    out_ref[...] = jax.nn.sigmoid(gT) * onw_ref[:, 0:1]


def _gatepath_T(hsTb, w1T, w2T, onwT):
    HID, S = hsTb.shape
    BN = 512
    return pl.pallas_call(
        _gatepath_kern,
        grid=(S // BN,),
        in_specs=[
            pl.BlockSpec((D, HID), lambda n: (0, 0)),
            pl.BlockSpec((HID, BN), lambda n: (0, n)),
            pl.BlockSpec((HID, D), lambda n: (0, 0)),
            pl.BlockSpec((HID, 128), lambda n: (0, 0)),
        ],
        out_specs=pl.BlockSpec((HID, BN), lambda n: (0, n)),
        out_shape=jax.ShapeDtypeStruct((HID, S), F32),
        compiler_params=pltpu.CompilerParams(
            dimension_semantics=("parallel",)),
    )(w1T, hsTb, w2T, onwT)


def _attn_kern(qT_ref, k_ref, vT_ref, sg_ref, tq_ref, tk_ref, out_ref, *,
               S, scale):
    C = S // CS
    qT = qT_ref[...]
    cq, sq = tq_ref[: D // 2, :], tq_ref[D // 2:, :]
    q1, q2 = qT[: D // 2, :], qT[D // 2:, :]
    qbT = jnp.concatenate([q1 * cq - q2 * sq, q2 * cq + q1 * sq],
                          axis=0).astype(BF)
    k = k_ref[:, 0, 0, :]
    ck, sk = tk_ref[:, : D // 2], tk_ref[:, D // 2:]
    k1, k2 = k[:, : D // 2], k[:, D // 2:]
    kr = jnp.concatenate([k1 * ck - k2 * sk, k2 * ck + k1 * sk], axis=1)
    kb = kr.astype(BF)

    kc = jnp.concatenate(
        [jnp.mean(kr[c * CS:(c + 1) * CS, :], axis=0, keepdims=True)
         for c in range(C)], axis=0)
    g = _dot(kc.astype(BF), qbT)
    c = jax.lax.broadcasted_iota(jnp.int32, (C, S), 0)
    pos = jax.lax.broadcasted_iota(jnp.int32, (C, S), 1)
    cid = pos // CS
    g = jnp.where(pos < (c + 1) * CS, -jnp.inf, g)
    g = jnp.where(cid == c, jnp.inf, g)
    cnt = jnp.zeros((C, S), jnp.int32)
    for cc in range(C):
        cnt = cnt + (g[cc:cc + 1, :] >= g).astype(jnp.int32)
    cand = jnp.where(cnt >= TOPK, g, -jnp.inf)
    thresh = jnp.max(cand, axis=0, keepdims=True)
    bias = jnp.where(g >= thresh, 0.0, NEG)

    ki = jax.lax.broadcasted_iota(jnp.int32, (CS, CS), 0)
    qi = jax.lax.broadcasted_iota(jnp.int32, (CS, CS), 1)
    tri = ki > qi

    for i in range(C):
        qT_i = qbT[:, i * CS:(i + 1) * CS]
        m = jnp.full((1, CS), NEG, dtype=F32)
        l = jnp.zeros((1, CS), dtype=F32)
        acc = jnp.zeros((D, CS), dtype=F32)
        for j in range(i + 1):
            kj = kb[j * CS:(j + 1) * CS, :]
            s = _dot(kj, qT_i) * scale
            s = s + bias[j:j + 1, i * CS:(i + 1) * CS]
            if j == i:
                s = jnp.where(tri, NEG, s)
            m_new = jnp.maximum(m, jnp.max(s, axis=0, keepdims=True))
            r = jnp.exp(m - m_new)
            p = jnp.exp(s - m_new)
            l = l * r + jnp.sum(p, axis=0, keepdims=True)
            vTj = vT_ref[:, j * CS:(j + 1) * CS]
            acc = acc * r + _dot(vTj, p.astype(BF))
            m = m_new
        oT = acc / l
        rms = jax.lax.rsqrt(jnp.mean(oT * oT, axis=0, keepdims=True) + EPS)
        out_ref[:, i * CS:(i + 1) * CS] = \
            (oT * rms * sg_ref[:, i * CS:(i + 1) * CS]).astype(BF)


def _attention(qT, k4, vT, sgT, tqT, tk):
    S = k4.shape[0]
    scale = 1.0 / (D ** 0.5)
    return pl.pallas_call(
        functools.partial(_attn_kern, S=S, scale=scale),
        grid=(H,),
        in_specs=[
            pl.BlockSpec((D, S), lambda h: (h, 0)),
            pl.BlockSpec((S, 1, 1, D), lambda h: (0, h, 0, 0)),
            pl.BlockSpec((D, S), lambda h: (h, 0)),
            pl.BlockSpec((D, S), lambda h: (h, 0)),
            pl.BlockSpec((D, S), lambda h: (0, 0)),
            pl.BlockSpec((S, D), lambda h: (0, 0)),
        ],
        out_specs=pl.BlockSpec((D, S), lambda h: (h, 0)),
        out_shape=jax.ShapeDtypeStruct((H * D, S), BF),
        compiler_params=pltpu.CompilerParams(
            dimension_semantics=("parallel",)),
    )(qT, k4, vT, sgT, tqT, tk)


def kernel(hidden_states, Wq, Wk, Wv, Wo, Wg1, Wg2, o_norm_w):
    B, S, HID = hidden_states.shape
    hs = hidden_states[0]
    hsb = hs.astype(BF)
    hsTb = hsb.T

    pos = jnp.arange(S)
    inv = 1.0 / (10000.0 ** (jnp.arange(0, D, 2, dtype=F32) / D))
    fr = pos[:, None].astype(F32) * inv[None, :]
    cos = jnp.cos(fr)
    sin = jnp.sin(fr)
    tk = jnp.concatenate([cos, sin], axis=1)
    tqT = tk.T.copy()

    qT = _matmul(Wq.astype(BF).T, hsTb, out_bf16=False)
    kN = _matmul(hsb, Wk.astype(BF), out_bf16=False, bn=1024)
    vT = _matmul(Wv.astype(BF).T, hsTb, out_bf16=True)
    onwT = jnp.broadcast_to(jnp.tile(o_norm_w, H)[:, None], (H * D, 128))
    sgT = _gatepath_T(hsTb, Wg1.astype(BF).T, Wg2.astype(BF).T, onwT)

    k4 = kN.reshape(S, H, 1, D)
    oT = _attention(qT, k4, vT, sgT, tqT, tk)

    outT = _matmul(Wo.astype(BF).T, oT, out_bf16=False)
    return outT.T[None]


# speedup vs baseline: 1.4596x; 1.1924x over previous
import functools

import jax
import jax.numpy as jnp
from jax.experimental import pallas as pl
from jax.experimental.pallas import tpu as pltpu

H = 32
D = 64
CS = 256
TOPK = 4
EPS = 1e-6
NEG = -1e30

BF = jnp.bfloat16
F32 = jnp.float32


def _dot(a, b):
    return jax.lax.dot_general(
        a, b, dimension_numbers=(((a.ndim - 1,), (0,)), ((), ())),
        preferred_element_type=F32)


def _mm_kern(a_ref, b_ref, out_ref, *, out_bf16):
    r = _dot(a_ref[...], b_ref[...])
    out_ref[...] = r.astype(BF) if out_bf16 else r


def _matmul(a, b, out_bf16, bm=512, bn=512):
    M, K = a.shape
    N = b.shape[1]
    return pl.pallas_call(
        functools.partial(_mm_kern, out_bf16=out_bf16),
        grid=(M // bm, N // bn),
        in_specs=[
            pl.BlockSpec((bm, K), lambda m, n: (m, 0)),
            pl.BlockSpec((K, bn), lambda m, n: (0, n)),
        ],
        out_specs=pl.BlockSpec((bm, bn), lambda m, n: (m, n)),
        out_shape=jax.ShapeDtypeStruct((M, N), BF if out_bf16 else F32),
        compiler_params=pltpu.CompilerParams(
            dimension_semantics=("parallel", "parallel")),
    )(a, b)


def _projT_rope_kern(w_ref, wp_ref, hsT_ref, tbl_ref, out_ref, *, bm):
    x = _dot(w_ref[...], hsT_ref[...])
    xp = _dot(wp_ref[...], hsT_ref[...])
    reps = bm // D
    cosA = jnp.concatenate([tbl_ref[:D, :]] * reps, axis=0)
    sinA = jnp.concatenate([tbl_ref[D:, :]] * reps, axis=0)
    out_ref[...] = (x * cosA + xp * sinA).astype(BF)


def _projT_rope(wT, wTp, hsTb, tblT):
    HD, HID = wT.shape
    S = hsTb.shape[1]
    bm, bn = 512, 512
    return pl.pallas_call(
        functools.partial(_projT_rope_kern, bm=bm),
        grid=(HD // bm, S // bn),
        in_specs=[
            pl.BlockSpec((bm, HID), lambda m, n: (m, 0)),
            pl.BlockSpec((bm, HID), lambda m, n: (m, 0)),
            pl.BlockSpec((HID, bn), lambda m, n: (0, n)),
            pl.BlockSpec((2 * D, bn), lambda m, n: (0, n)),
        ],
        out_specs=pl.BlockSpec((bm, bn), lambda m, n: (m, n)),
        out_shape=jax.ShapeDtypeStruct((HD, S), BF),
        compiler_params=pltpu.CompilerParams(
            dimension_semantics=("parallel", "parallel")),
    )(wT, wTp, hsTb, tblT)


def _projN_rope_kern(hs_ref, w_ref, wp_ref, cos_ref, sin_ref, out_ref):
    x = _dot(hs_ref[...], w_ref[...])
    xp = _dot(hs_ref[...], wp_ref[...])
    out_ref[...] = x * cos_ref[...] + xp * sin_ref[...]


def _projN_rope(hsb, w, wp, cosT, sinT2):
    S, HID = hsb.shape
    HD = w.shape[1]
    bn = 512
    return pl.pallas_call(
        _projN_rope_kern,
        grid=(HD // bn,),
        in_specs=[
            pl.BlockSpec((S, HID), lambda n: (0, 0)),
            pl.BlockSpec((HID, bn), lambda n: (0, n)),
            pl.BlockSpec((HID, bn), lambda n: (0, n)),
            pl.BlockSpec((S, bn), lambda n: (0, n)),
            pl.BlockSpec((S, bn), lambda n: (0, n)),
        ],
        out_specs=pl.BlockSpec((S, bn), lambda n: (0, n)),
        out_shape=jax.ShapeDtypeStruct((S, HD), F32),
        compiler_params=pltpu.CompilerParams(
            dimension_semantics=("parallel",)),
    )(hsb, w, wp, cosT, sinT2)


def _gatepath_kern(w1T_ref, hsT_ref, w2T_ref, onw_ref, out_ref):
    tT = _dot(w1T_ref[...], hsT_ref[...]).astype(BF)
    gT = _dot(w2T_ref[...], tT)
    out_ref[...] = jax.nn.sigmoid(gT) * onw_ref[:, 0:1]


def _gatepath_T(hsTb, w1T, w2T, onwT):
    HID, S = hsTb.shape
    BN = 512
    return pl.pallas_call(
        _gatepath_kern,
        grid=(S // BN,),
        in_specs=[
            pl.BlockSpec((D, HID), lambda n: (0, 0)),
            pl.BlockSpec((HID, BN), lambda n: (0, n)),
            pl.BlockSpec((HID, D), lambda n: (0, 0)),
            pl.BlockSpec((HID, 128), lambda n: (0, 0)),
        ],
        out_specs=pl.BlockSpec((HID, BN), lambda n: (0, n)),
        out_shape=jax.ShapeDtypeStruct((HID, S), F32),
        compiler_params=pltpu.CompilerParams(
            dimension_semantics=("parallel",)),
    )(w1T, hsTb, w2T, onwT)


def _attn_kern(qT_ref, k_ref, vT_ref, sg_ref, out_ref, *, S, scale):
    C = S // CS
    qbT = qT_ref[...]
    k = k_ref[:, 0, 0, :]
    kb = k.astype(BF)

    kc = jnp.concatenate(
        [jnp.mean(k[c * CS:(c + 1) * CS, :], axis=0, keepdims=True)
         for c in range(C)], axis=0)
    g = _dot(kc.astype(BF), qbT)
    c = jax.lax.broadcasted_iota(jnp.int32, (C, S), 0)
    pos = jax.lax.broadcasted_iota(jnp.int32, (C, S), 1)
    cid = pos // CS
    g = jnp.where(pos < (c + 1) * CS, -jnp.inf, g)
    g = jnp.where(cid == c, jnp.inf, g)
    cnt = jnp.zeros((C, S), jnp.int32)
    for cc in range(C):
        cnt = cnt + (g[cc:cc + 1, :] >= g).astype(jnp.int32)
    cand = jnp.where(cnt >= TOPK, g, -jnp.inf)
    thresh = jnp.max(cand, axis=0, keepdims=True)
    bias = jnp.where(g >= thresh, 0.0, NEG)

    ki = jax.lax.broadcasted_iota(jnp.int32, (CS, CS), 0)
    qi = jax.lax.broadcasted_iota(jnp.int32, (CS, CS), 1)
    tri = ki > qi

    for i in range(C):
        qT_i = qbT[:, i * CS:(i + 1) * CS]
        m = jnp.full((1, CS), NEG, dtype=F32)
        l = jnp.zeros((1, CS), dtype=F32)
        acc = jnp.zeros((D, CS), dtype=F32)
        for j in range(i + 1):
            kj = kb[j * CS:(j + 1) * CS, :]
            s = _dot(kj, qT_i) * scale
            s = s + bias[j:j + 1, i * CS:(i + 1) * CS]
            if j == i:
                s = jnp.where(tri, NEG, s)
            m_new = jnp.maximum(m, jnp.max(s, axis=0, keepdims=True))
            r = jnp.exp(m - m_new)
            p = jnp.exp(s - m_new)
            l = l * r + jnp.sum(p, axis=0, keepdims=True)
            vTj = vT_ref[:, j * CS:(j + 1) * CS]
            acc = acc * r + _dot(vTj, p.astype(BF))
            m = m_new
        oT = acc / l
        rms = jax.lax.rsqrt(jnp.mean(oT * oT, axis=0, keepdims=True) + EPS)
        out_ref[:, i * CS:(i + 1) * CS] = \
            (oT * rms * sg_ref[:, i * CS:(i + 1) * CS]).astype(BF)


def _attention(qT, k4, vT, sgT):
    S = k4.shape[0]
    scale = 1.0 / (D ** 0.5)
    return pl.pallas_call(
        functools.partial(_attn_kern, S=S, scale=scale),
        grid=(H,),
        in_specs=[
            pl.BlockSpec((D, S), lambda h: (h, 0)),
            pl.BlockSpec((S, 1, 1, D), lambda h: (0, h, 0, 0)),
            pl.BlockSpec((D, S), lambda h: (h, 0)),
            pl.BlockSpec((D, S), lambda h: (h, 0)),
        ],
        out_specs=pl.BlockSpec((D, S), lambda h: (h, 0)),
        out_shape=jax.ShapeDtypeStruct((H * D, S), BF),
        compiler_params=pltpu.CompilerParams(
            dimension_semantics=("parallel",)),
    )(qT, k4, vT, sgT)


def _swap_cols(w):
    HID = w.shape[0]
    w3 = w.reshape(HID, H, D)
    return jnp.concatenate([w3[:, :, D // 2:], w3[:, :, : D // 2]],
                           axis=-1).reshape(HID, H * D)


def kernel(hidden_states, Wq, Wk, Wv, Wo, Wg1, Wg2, o_norm_w):
    B, S, HID = hidden_states.shape
    hs = hidden_states[0]
    hsb = hs.astype(BF)
    hsTb = hsb.T

    pos = jnp.arange(S)
    inv = 1.0 / (10000.0 ** (jnp.arange(0, D, 2, dtype=F32) / D))
    fr = pos[:, None].astype(F32) * inv[None, :]
    cos = jnp.cos(fr)
    sin = jnp.sin(fr)
    cosA = jnp.concatenate([cos.T, cos.T], axis=0)
    sinA = jnp.concatenate([-sin.T, sin.T], axis=0)
    tblT = jnp.concatenate([cosA, sinA], axis=0)
    cosT = jnp.tile(jnp.concatenate([cos, cos], axis=1), (1, H))
    sinT2 = jnp.tile(jnp.concatenate([-sin, sin], axis=1), (1, H))

    WqT = Wq.astype(BF).T
    WqTp = _swap_cols(Wq).astype(BF).T
    qT = _projT_rope(WqT, WqTp, hsTb, tblT)
    kN = _projN_rope(hsb, Wk.astype(BF), _swap_cols(Wk).astype(BF),
                     cosT, sinT2)
    vT = _matmul(Wv.astype(BF).T, hsTb, out_bf16=True)
    onwT = jnp.broadcast_to(jnp.tile(o_norm_w, H)[:, None], (H * D, 128))
    sgT = _gatepath_T(hsTb, Wg1.astype(BF).T, Wg2.astype(BF).T, onwT)

    k4 = kN.reshape(S, H, 1, D)
    oT = _attention(qT, k4, vT, sgT)

    outT = _matmul(Wo.astype(BF).T, oT, out_bf16=False)
    return outT.T[None]


# speedup vs baseline: 1.7528x; 1.2009x over previous
import functools

import jax
import jax.numpy as jnp
from jax.experimental import pallas as pl
from jax.experimental.pallas import tpu as pltpu

H = 32
D = 64
CS = 256
TOPK = 4
EPS = 1e-6
NEG = -1e30

BF = jnp.bfloat16
F32 = jnp.float32


def _dot(a, b):
    return jax.lax.dot_general(
        a, b, dimension_numbers=(((a.ndim - 1,), (0,)), ((), ())),
        preferred_element_type=F32)


def _mm_kern(a_ref, b_ref, out_ref, *, out_bf16):
    r = _dot(a_ref[...], b_ref[...])
    out_ref[...] = r.astype(BF) if out_bf16 else r


def _matmul(a, b, out_bf16, bm=512, bn=512):
    M, K = a.shape
    N = b.shape[1]
    return pl.pallas_call(
        functools.partial(_mm_kern, out_bf16=out_bf16),
        grid=(M // bm, N // bn),
        in_specs=[
            pl.BlockSpec((bm, K), lambda m, n: (m, 0)),
            pl.BlockSpec((K, bn), lambda m, n: (0, n)),
        ],
        out_specs=pl.BlockSpec((bm, bn), lambda m, n: (m, n)),
        out_shape=jax.ShapeDtypeStruct((M, N), BF if out_bf16 else F32),
        compiler_params=pltpu.CompilerParams(
            dimension_semantics=("parallel", "parallel")),
    )(a, b)


def _projT_rope_kern(w_ref, wp_ref, hsT_ref, tbl_ref, out_ref, *, bm):
    x = _dot(w_ref[...], hsT_ref[...])
    xp = _dot(wp_ref[...], hsT_ref[...])
    reps = bm // D
    cosA = jnp.concatenate([tbl_ref[:D, :]] * reps, axis=0)
    sinA = jnp.concatenate([tbl_ref[D:, :]] * reps, axis=0)
    out_ref[...] = (x * cosA + xp * sinA).astype(BF)


def _projT_rope(wT, wTp, hsTb, tblT):
    HD, HID = wT.shape
    S = hsTb.shape[1]
    bm, bn = 512, 512
    return pl.pallas_call(
        functools.partial(_projT_rope_kern, bm=bm),
        grid=(HD // bm, S // bn),
        in_specs=[
            pl.BlockSpec((bm, HID), lambda m, n: (m, 0)),
            pl.BlockSpec((bm, HID), lambda m, n: (m, 0)),
            pl.BlockSpec((HID, bn), lambda m, n: (0, n)),
            pl.BlockSpec((2 * D, bn), lambda m, n: (0, n)),
        ],
        out_specs=pl.BlockSpec((bm, bn), lambda m, n: (m, n)),
        out_shape=jax.ShapeDtypeStruct((HD, S), BF),
        compiler_params=pltpu.CompilerParams(
            dimension_semantics=("parallel", "parallel")),
    )(wT, wTp, hsTb, tblT)


def _projN_rope_kern(hs_ref, w_ref, wp_ref, cos_ref, sin_ref, out_ref):
    x = _dot(hs_ref[...], w_ref[...])
    xp = _dot(hs_ref[...], wp_ref[...])
    out_ref[...] = x * cos_ref[...] + xp * sin_ref[...]


def _projN_rope(hsb, w, wp, cosT, sinT2):
    S, HID = hsb.shape
    HD = w.shape[1]
    bn = 512
    return pl.pallas_call(
        _projN_rope_kern,
        grid=(HD // bn,),
        in_specs=[
            pl.BlockSpec((S, HID), lambda n: (0, 0)),
            pl.BlockSpec((HID, bn), lambda n: (0, n)),
            pl.BlockSpec((HID, bn), lambda n: (0, n)),
            pl.BlockSpec((S, bn), lambda n: (0, n)),
            pl.BlockSpec((S, bn), lambda n: (0, n)),
        ],
        out_specs=pl.BlockSpec((S, bn), lambda n: (0, n)),
        out_shape=jax.ShapeDtypeStruct((S, HD), F32),
        compiler_params=pltpu.CompilerParams(
            dimension_semantics=("parallel",)),
    )(hsb, w, wp, cosT, sinT2)


def _gatepath_kern(w1T_ref, hsT_ref, w2T_ref, onw_ref, out_ref):
    tT = _dot(w1T_ref[...], hsT_ref[...]).astype(BF)
    gT = _dot(w2T_ref[...], tT)
    out_ref[...] = jax.nn.sigmoid(gT) * onw_ref[:, 0:1]


def _gatepath_T(hsTb, w1T, w2T, onwT):
    HID, S = hsTb.shape
    BN = 512
    return pl.pallas_call(
        _gatepath_kern,
        grid=(S // BN,),
        in_specs=[
            pl.BlockSpec((D, HID), lambda n: (0, 0)),
            pl.BlockSpec((HID, BN), lambda n: (0, n)),
            pl.BlockSpec((HID, D), lambda n: (0, 0)),
            pl.BlockSpec((HID, 128), lambda n: (0, 0)),
        ],
        out_specs=pl.BlockSpec((HID, BN), lambda n: (0, n)),
        out_shape=jax.ShapeDtypeStruct((HID, S), F32),
        compiler_params=pltpu.CompilerParams(
            dimension_semantics=("parallel",)),
    )(w1T, hsTb, w2T, onwT)


def _attn_kern(qT_ref, k_ref, vT_ref, sg_ref, out_ref, *, S, scale):
    C = S // CS
    qbT = qT_ref[...]
    k = k_ref[:, 0, 0, :]
    kb = k.astype(BF)

    kc = jnp.concatenate(
        [jnp.mean(k[c * CS:(c + 1) * CS, :], axis=0, keepdims=True)
         for c in range(C)], axis=0)
    g = _dot(kc.astype(BF), qbT)
    c = jax.lax.broadcasted_iota(jnp.int32, (C, S), 0)
    pos = jax.lax.broadcasted_iota(jnp.int32, (C, S), 1)
    cid = pos // CS
    g = jnp.where(pos < (c + 1) * CS, -jnp.inf, g)
    g = jnp.where(cid == c, jnp.inf, g)
    cnt = jnp.zeros((C, S), jnp.int32)
    for cc in range(C):
        cnt = cnt + (g[cc:cc + 1, :] >= g).astype(jnp.int32)
    cand = jnp.where(cnt >= TOPK, g, -jnp.inf)
    thresh = jnp.max(cand, axis=0, keepdims=True)
    bias = jnp.where(g >= thresh, 0.0, NEG)

    ki = jax.lax.broadcasted_iota(jnp.int32, (CS, CS), 0)
    qi = jax.lax.broadcasted_iota(jnp.int32, (CS, CS), 1)
    tri = ki > qi

    for i in range(C):
        L = (i + 1) * CS
        qT_i = qbT[:, i * CS:(i + 1) * CS]
        s = _dot(kb[:L, :], qT_i) * scale
        be = jax.lax.broadcast_in_dim(
            bias[:i + 1, i * CS:(i + 1) * CS], (i + 1, CS, CS), (0, 2))
        s = s + be.reshape(L, CS)
        s_self = jnp.where(tri, NEG, s[i * CS:, :])
        if i > 0:
            s = jnp.concatenate([s[:i * CS, :], s_self], axis=0)
        else:
            s = s_self
        m = jnp.max(s, axis=0, keepdims=True)
        p = jnp.exp(s - m)
        l = jnp.sum(p, axis=0, keepdims=True)
        oT = _dot(vT_ref[:, :L], p.astype(BF)) / l
        rms = jax.lax.rsqrt(jnp.mean(oT * oT, axis=0, keepdims=True) + EPS)
        out_ref[:, i * CS:(i + 1) * CS] = \
            (oT * rms * sg_ref[:, i * CS:(i + 1) * CS]).astype(BF)


def _attention(qT, k4, vT, sgT):
    S = k4.shape[0]
    scale = 1.0 / (D ** 0.5)
    return pl.pallas_call(
        functools.partial(_attn_kern, S=S, scale=scale),
        grid=(H,),
        in_specs=[
            pl.BlockSpec((D, S), lambda h: (h, 0)),
            pl.BlockSpec((S, 1, 1, D), lambda h: (0, h, 0, 0)),
            pl.BlockSpec((D, S), lambda h: (h, 0)),
            pl.BlockSpec((D, S), lambda h: (h, 0)),
        ],
        out_specs=pl.BlockSpec((D, S), lambda h: (h, 0)),
        out_shape=jax.ShapeDtypeStruct((H * D, S), BF),
        compiler_params=pltpu.CompilerParams(
            dimension_semantics=("parallel",)),
    )(qT, k4, vT, sgT)


def _swap_cols(w):
    HID = w.shape[0]
    w3 = w.reshape(HID, H, D)
    return jnp.concatenate([w3[:, :, D // 2:], w3[:, :, : D // 2]],
                           axis=-1).reshape(HID, H * D)


def kernel(hidden_states, Wq, Wk, Wv, Wo, Wg1, Wg2, o_norm_w):
    B, S, HID = hidden_states.shape
    hs = hidden_states[0]
    hsb = hs.astype(BF)
    hsTb = hsb.T

    pos = jnp.arange(S)
    inv = 1.0 / (10000.0 ** (jnp.arange(0, D, 2, dtype=F32) / D))
    fr = pos[:, None].astype(F32) * inv[None, :]
    cos = jnp.cos(fr)
    sin = jnp.sin(fr)
    cosA = jnp.concatenate([cos.T, cos.T], axis=0)
    sinA = jnp.concatenate([-sin.T, sin.T], axis=0)
    tblT = jnp.concatenate([cosA, sinA], axis=0)
    cosT = jnp.tile(jnp.concatenate([cos, cos], axis=1), (1, H))
    sinT2 = jnp.tile(jnp.concatenate([-sin, sin], axis=1), (1, H))

    WqT = Wq.astype(BF).T
    WqTp = _swap_cols(Wq).astype(BF).T
    qT = _projT_rope(WqT, WqTp, hsTb, tblT)
    kN = _projN_rope(hsb, Wk.astype(BF), _swap_cols(Wk).astype(BF),
                     cosT, sinT2)
    vT = _matmul(Wv.astype(BF).T, hsTb, out_bf16=True)
    onwT = jnp.broadcast_to(jnp.tile(o_norm_w, H)[:, None], (H * D, 128))
    sgT = _gatepath_T(hsTb, Wg1.astype(BF).T, Wg2.astype(BF).T, onwT)

    k4 = kN.reshape(S, H, 1, D)
    oT = _attention(qT, k4, vT, sgT)

    outT = _matmul(Wo.astype(BF).T, oT, out_bf16=False)
    return outT.T[None]


# speedup vs baseline: 2.0288x; 1.1575x over previous
import functools

import jax
import jax.numpy as jnp
from jax.experimental import pallas as pl
from jax.experimental.pallas import tpu as pltpu

H = 32
D = 64
CS = 256
TOPK = 4
EPS = 1e-6
NEG = -1e30

BF = jnp.bfloat16
F32 = jnp.float32


def _dot(a, b):
    return jax.lax.dot_general(
        a, b, dimension_numbers=(((a.ndim - 1,), (0,)), ((), ())),
        preferred_element_type=F32)


def _mm_kern(a_ref, b_ref, out_ref, *, out_bf16):
    r = _dot(a_ref[...], b_ref[...])
    out_ref[...] = r.astype(BF) if out_bf16 else r


def _matmul(a, b, out_bf16, bm=512, bn=512):
    M, K = a.shape
    N = b.shape[1]
    return pl.pallas_call(
        functools.partial(_mm_kern, out_bf16=out_bf16),
        grid=(M // bm, N // bn),
        in_specs=[
            pl.BlockSpec((bm, K), lambda m, n: (m, 0)),
            pl.BlockSpec((K, bn), lambda m, n: (0, n)),
        ],
        out_specs=pl.BlockSpec((bm, bn), lambda m, n: (m, n)),
        out_shape=jax.ShapeDtypeStruct((M, N), BF if out_bf16 else F32),
        compiler_params=pltpu.CompilerParams(
            dimension_semantics=("parallel", "parallel")),
    )(a, b)


def _projT_rope_kern(w_ref, wp_ref, hsT_ref, tbl_ref, out_ref, *, bm):
    x = _dot(w_ref[...], hsT_ref[...])
    xp = _dot(wp_ref[...], hsT_ref[...])
    reps = bm // D
    cosA = jnp.concatenate([tbl_ref[:D, :]] * reps, axis=0)
    sinA = jnp.concatenate([tbl_ref[D:, :]] * reps, axis=0)
    out_ref[...] = (x * cosA + xp * sinA).astype(BF)


def _projT_rope(wT, wTp, hsTb, tblT):
    HD, HID = wT.shape
    S = hsTb.shape[1]
    bm, bn = 512, 512
    return pl.pallas_call(
        functools.partial(_projT_rope_kern, bm=bm),
        grid=(HD // bm, S // bn),
        in_specs=[
            pl.BlockSpec((bm, HID), lambda m, n: (m, 0)),
            pl.BlockSpec((bm, HID), lambda m, n: (m, 0)),
            pl.BlockSpec((HID, bn), lambda m, n: (0, n)),
            pl.BlockSpec((2 * D, bn), lambda m, n: (0, n)),
        ],
        out_specs=pl.BlockSpec((bm, bn), lambda m, n: (m, n)),
        out_shape=jax.ShapeDtypeStruct((HD, S), BF),
        compiler_params=pltpu.CompilerParams(
            dimension_semantics=("parallel", "parallel")),
    )(wT, wTp, hsTb, tblT)


def _projN_rope_kern(hs_ref, w_ref, wp_ref, cos_ref, sin_ref, out_ref):
    x = _dot(hs_ref[...], w_ref[...])
    xp = _dot(hs_ref[...], wp_ref[...])
    out_ref[...] = x * cos_ref[...] + xp * sin_ref[...]


def _projN_rope(hsb, w, wp, cosT, sinT2):
    S, HID = hsb.shape
    HD = w.shape[1]
    bn = 512
    return pl.pallas_call(
        _projN_rope_kern,
        grid=(HD // bn,),
        in_specs=[
            pl.BlockSpec((S, HID), lambda n: (0, 0)),
            pl.BlockSpec((HID, bn), lambda n: (0, n)),
            pl.BlockSpec((HID, bn), lambda n: (0, n)),
            pl.BlockSpec((S, bn), lambda n: (0, n)),
            pl.BlockSpec((S, bn), lambda n: (0, n)),
        ],
        out_specs=pl.BlockSpec((S, bn), lambda n: (0, n)),
        out_shape=jax.ShapeDtypeStruct((S, HD), F32),
        compiler_params=pltpu.CompilerParams(
            dimension_semantics=("parallel",)),
    )(hsb, w, wp, cosT, sinT2)


def _gatepath_kern(w1T_ref, hsT_ref, w2T_ref, onw_ref, out_ref):
    tT = _dot(w1T_ref[...], hsT_ref[...]).astype(BF)
    gT = _dot(w2T_ref[...], tT)
    out_ref[...] = (jax.nn.sigmoid(gT) * onw_ref[:, 0:1]).astype(BF)


def _gatepath_T(hsTb, w1T, w2T, onwT):
    HID, S = hsTb.shape
    BN = 512
    return pl.pallas_call(
        _gatepath_kern,
        grid=(S // BN,),
        in_specs=[
            pl.BlockSpec((D, HID), lambda n: (0, 0)),
            pl.BlockSpec((HID, BN), lambda n: (0, n)),
            pl.BlockSpec((HID, D), lambda n: (0, 0)),
            pl.BlockSpec((HID, 128), lambda n: (0, 0)),
        ],
        out_specs=pl.BlockSpec((HID, BN), lambda n: (0, n)),
        out_shape=jax.ShapeDtypeStruct((HID, S), BF),
        compiler_params=pltpu.CompilerParams(
            dimension_semantics=("parallel",)),
    )(w1T, hsTb, w2T, onwT)


def _attn_kern(qT_ref, k_ref, vT_ref, sg_ref, out_ref, *, S, scale):
    C = S // CS
    qbT = qT_ref[...]
    k = k_ref[0]
    kb = k.astype(BF)

    kc = jnp.concatenate(
        [jnp.mean(k[c * CS:(c + 1) * CS, :], axis=0, keepdims=True)
         for c in range(C)], axis=0)
    g = _dot(kc.astype(BF), qbT)
    c = jax.lax.broadcasted_iota(jnp.int32, (C, S), 0)
    pos = jax.lax.broadcasted_iota(jnp.int32, (C, S), 1)
    cid = pos // CS
    g = jnp.where(pos < (c + 1) * CS, -jnp.inf, g)
    g = jnp.where(cid == c, jnp.inf, g)
    cnt = jnp.zeros((C, S), jnp.int32)
    for cc in range(C):
        cnt = cnt + (g[cc:cc + 1, :] >= g).astype(jnp.int32)
    cand = jnp.where(cnt >= TOPK, g, -jnp.inf)
    thresh = jnp.max(cand, axis=0, keepdims=True)
    bias = jnp.where(g >= thresh, 0.0, NEG)

    ki = jax.lax.broadcasted_iota(jnp.int32, (CS, CS), 0)
    qi = jax.lax.broadcasted_iota(jnp.int32, (CS, CS), 1)
    tri = ki > qi

    for i in range(C):
        L = (i + 1) * CS
        qT_i = qbT[:, i * CS:(i + 1) * CS]
        s = _dot(kb[:L, :], qT_i) * scale
        be = jax.lax.broadcast_in_dim(
            bias[:i + 1, i * CS:(i + 1) * CS], (i + 1, CS, CS), (0, 2))
        s = s + be.reshape(L, CS)
        s_self = jnp.where(tri, NEG, s[i * CS:, :])
        if i > 0:
            s = jnp.concatenate([s[:i * CS, :], s_self], axis=0)
        else:
            s = s_self
        m = jnp.max(s, axis=0, keepdims=True)
        p = jnp.exp(s - m)
        l = jnp.sum(p, axis=0, keepdims=True)
        oT = _dot(vT_ref[:, :L], p.astype(BF)) / l
        rms = jax.lax.rsqrt(jnp.mean(oT * oT, axis=0, keepdims=True) + EPS)
        out_ref[:, i * CS:(i + 1) * CS] = \
            (oT * rms * sg_ref[:, i * CS:(i + 1) * CS]).astype(BF)


def _attention(qT, k3, vT, sgT):
    S = k3.shape[1]
    scale = 1.0 / (D ** 0.5)
    return pl.pallas_call(
        functools.partial(_attn_kern, S=S, scale=scale),
        grid=(H,),
        in_specs=[
            pl.BlockSpec((D, S), lambda h: (h, 0)),
            pl.BlockSpec((1, S, D), lambda h: (h, 0, 0)),
            pl.BlockSpec((D, S), lambda h: (h, 0)),
            pl.BlockSpec((D, S), lambda h: (h, 0)),
        ],
        out_specs=pl.BlockSpec((D, S), lambda h: (h, 0)),
        out_shape=jax.ShapeDtypeStruct((H * D, S), BF),
        compiler_params=pltpu.CompilerParams(
            dimension_semantics=("parallel",)),
    )(qT, k3, vT, sgT)


def _swap_cols(w):
    HID = w.shape[0]
    w3 = w.reshape(HID, H, D)
    return jnp.concatenate([w3[:, :, D // 2:], w3[:, :, : D // 2]],
                           axis=-1).reshape(HID, H * D)


def kernel(hidden_states, Wq, Wk, Wv, Wo, Wg1, Wg2, o_norm_w):
    B, S, HID = hidden_states.shape
    hs = hidden_states[0]
    hsb = hs.astype(BF)
    hsTb = hsb.T

    pos = jnp.arange(S)
    inv = 1.0 / (10000.0 ** (jnp.arange(0, D, 2, dtype=F32) / D))
    fr = pos[:, None].astype(F32) * inv[None, :]
    cos = jnp.cos(fr)
    sin = jnp.sin(fr)
    cosA = jnp.concatenate([cos.T, cos.T], axis=0)
    sinA = jnp.concatenate([-sin.T, sin.T], axis=0)
    tblT = jnp.concatenate([cosA, sinA], axis=0)
    cosT = jnp.tile(jnp.concatenate([cos, cos], axis=1), (1, H))
    sinT2 = jnp.tile(jnp.concatenate([-sin, sin], axis=1), (1, H))

    WqT = Wq.astype(BF).T
    WqTp = _swap_cols(Wq).astype(BF).T
    qT = _projT_rope(WqT, WqTp, hsTb, tblT)
    kN = _projN_rope(hsb, Wk.astype(BF), _swap_cols(Wk).astype(BF),
                     cosT, sinT2)
    vT = _matmul(Wv.astype(BF).T, hsTb, out_bf16=True)
    onwT = jnp.broadcast_to(jnp.tile(o_norm_w, H)[:, None], (H * D, 128))
    sgT = _gatepath_T(hsTb, Wg1.astype(BF).T, Wg2.astype(BF).T, onwT)

    k3 = kN.reshape(S, H, D).transpose(1, 0, 2)
    oT = _attention(qT, k3, vT, sgT)

    outT = _matmul(Wo.astype(BF).T, oT, out_bf16=False)
    return outT.T[None]
